# baseline (device time: 490997 ns/iter reference)
import jax
import jax.numpy as jnp
from jax import lax
from jax.experimental import pallas as pl
from jax.experimental.pallas import tpu as pltpu

NZ = 4
T = 2048
D = 1024
EXP_PER = 4
CAP = 768
AROWS = T // 128


def _ag_body(x_ref, a_ref, ox_ref, oa_ref, sx, rx, sa, ra):
    my_x = lax.axis_index("x")
    my_y = lax.axis_index("y")
    my_z = lax.axis_index("z")

    barrier = pltpu.get_barrier_semaphore()
    for o in range(1, NZ):
        pl.semaphore_signal(
            barrier, inc=1,
            device_id=(my_x, my_y, (my_z + o) % NZ),
            device_id_type=pl.DeviceIdType.MESH,
        )
    pl.semaphore_wait(barrier, NZ - 1)

    ox_ref[pl.ds(my_z * T, T), :] = x_ref[...]
    oa_ref[pl.ds(my_z * AROWS, AROWS), :] = a_ref[...]

    descs = []
    for o in range(1, NZ):
        peer = (my_z + o) % NZ
        dev = (my_x, my_y, peer)
        dx = pltpu.make_async_remote_copy(
            src_ref=x_ref,
            dst_ref=ox_ref.at[pl.ds(my_z * T, T)],
            send_sem=sx.at[o - 1],
            recv_sem=rx.at[o - 1],
            device_id=dev,
            device_id_type=pl.DeviceIdType.MESH,
        )
        da = pltpu.make_async_remote_copy(
            src_ref=a_ref,
            dst_ref=oa_ref.at[pl.ds(my_z * AROWS, AROWS)],
            send_sem=sa.at[o - 1],
            recv_sem=ra.at[o - 1],
            device_id=dev,
            device_id_type=pl.DeviceIdType.MESH,
        )
        dx.start()
        da.start()
        descs.append((dx, da))
    for dx, da in descs:
        dx.wait()
        da.wait()


def _rs_body(p_ref, o_ref, comm, ss, rs):
    my_x = lax.axis_index("x")
    my_y = lax.axis_index("y")
    my_z = lax.axis_index("z")

    barrier = pltpu.get_barrier_semaphore()
    for o in range(1, NZ):
        pl.semaphore_signal(
            barrier, inc=1,
            device_id=(my_x, my_y, (my_z + o) % NZ),
            device_id_type=pl.DeviceIdType.MESH,
        )
    pl.semaphore_wait(barrier, NZ - 1)

    descs = []
    for o in range(1, NZ):
        peer = (my_z + o) % NZ
        d = pltpu.make_async_remote_copy(
            src_ref=p_ref.at[pl.ds(peer * T, T)],
            dst_ref=comm.at[o - 1],
            send_sem=ss.at[o - 1],
            recv_sem=rs.at[o - 1],
            device_id=(my_x, my_y, peer),
            device_id_type=pl.DeviceIdType.MESH,
        )
        d.start()
        descs.append(d)
    for d in descs:
        d.wait()

    acc = p_ref[pl.ds(my_z * T, T), :].astype(jnp.float32)
    for k in range(NZ - 1):
        acc = acc + comm[k].astype(jnp.float32)
    o_ref[...] = acc


def kernel(x, assign, W1, W2):
    x_bf = x.astype(jnp.bfloat16)
    a2d = assign.reshape(AROWS, 128)

    x_all, a_all2d = pl.pallas_call(
        _ag_body,
        out_shape=(
            jax.ShapeDtypeStruct((NZ * T, D), jnp.bfloat16),
            jax.ShapeDtypeStruct((NZ * AROWS, 128), jnp.int32),
        ),
        in_specs=[
            pl.BlockSpec(memory_space=pltpu.VMEM),
            pl.BlockSpec(memory_space=pltpu.VMEM),
        ],
        out_specs=(
            pl.BlockSpec(memory_space=pltpu.VMEM),
            pl.BlockSpec(memory_space=pltpu.VMEM),
        ),
        scratch_shapes=[
            pltpu.SemaphoreType.DMA((NZ - 1,)),
            pltpu.SemaphoreType.DMA((NZ - 1,)),
            pltpu.SemaphoreType.DMA((NZ - 1,)),
            pltpu.SemaphoreType.DMA((NZ - 1,)),
        ],
        compiler_params=pltpu.CompilerParams(collective_id=0),
    )(x_bf, a2d)

    a_all = a_all2d.reshape(NZ * T)

    my_z = lax.axis_index("z")
    sa, order = lax.sort_key_val(a_all, jnp.arange(NZ * T, dtype=jnp.int32))
    experts = my_z * EXP_PER + jnp.arange(EXP_PER, dtype=jnp.int32)
    starts = jnp.searchsorted(sa, experts)
    idx = jax.vmap(lambda s: lax.dynamic_slice(order, (s,), (CAP,)))(starts)
    sval = jax.vmap(lambda s: lax.dynamic_slice(sa, (s,), (CAP,)))(starts)
    valid = sval == experts[:, None]

    xg = jnp.take(x_all, idx.reshape(-1), axis=0).reshape(EXP_PER, CAP, D)
    w1 = W1.astype(jnp.bfloat16)
    w2 = W2.astype(jnp.bfloat16)
    h = jax.nn.relu(
        jnp.einsum("gcd,gdf->gcf", xg, w1, preferred_element_type=jnp.float32)
    ).astype(jnp.bfloat16)
    y = jnp.einsum("gcf,gfd->gcd", h, w2, preferred_element_type=jnp.float32)
    y = jnp.where(valid[..., None], y, 0.0).astype(jnp.bfloat16)

    idx_safe = jnp.where(valid, idx, NZ * T)
    partial = (
        jnp.zeros((NZ * T, D), jnp.bfloat16)
        .at[idx_safe.reshape(-1)]
        .set(y.reshape(-1, D), mode="drop")
    )

    return pl.pallas_call(
        _rs_body,
        out_shape=jax.ShapeDtypeStruct((T, D), jnp.float32),
        in_specs=[pl.BlockSpec(memory_space=pltpu.VMEM)],
        out_specs=pl.BlockSpec(memory_space=pltpu.VMEM),
        scratch_shapes=[
            pltpu.VMEM((NZ - 1, T, D), jnp.bfloat16),
            pltpu.SemaphoreType.DMA((NZ - 1,)),
            pltpu.SemaphoreType.DMA((NZ - 1,)),
        ],
        compiler_params=pltpu.CompilerParams(collective_id=1),
    )(partial)
